# baseline (speedup 1.0000x reference)
import jax
import jax.numpy as jnp
from jax import lax
from jax.experimental import pallas as pl
from jax.experimental.pallas import tpu as pltpu
from jax.experimental.pallas import tpu_sc as plsc

NU = 4000
NI = 6000
NN = NU + NI
NE = 320000
D = 128
NLAYER = 3

NC = 2
NS = 16
NW = NC * NS
CH = 80
EPT = 10240
EPAD = NW * EPT
NCHUNK = EPT // CH
NP = 10240
RPT = NP // NS


def _sc_layer_body(x_hbm, packed_hbm, w_hbm, part_hbm,
                   r0_v, r1_v, r2_v, r3_v,
                   p0_v, p1_v, p2_v, p3_v,
                   w0_v, w1_v, w2_v, w3_v,
                   d0_v, d1_v, d2_v, d3_v, acc,
                   g0, g1, g2, g3, f0, f1, f2, f3):
    cid = lax.axis_index("c")
    sid = lax.axis_index("s")
    rows = (r0_v, r1_v, r2_v, r3_v)
    pidx = (p0_v, p1_v, p2_v, p3_v)
    wring = (w0_v, w1_v, w2_v, w3_v)
    didx = (d0_v, d1_v, d2_v, d3_v)
    gsem = (g0, g1, g2, g3)
    fsem = (f0, f1, f2, f3)
    wid = cid * NS + sid
    cbase = wid * NCHUNK

    def sync_fetch(c, b):
        pltpu.sync_copy(packed_hbm.at[pl.ds(cbase + c, 1)], pidx[b])
        pltpu.sync_copy(w_hbm.at[pl.ds(cbase + c, 1)], wring[b])

    def start_gather(b):
        pltpu.async_copy(x_hbm.at[pidx[b].at[0, 0]], rows[b], gsem[b])

    def wait_gather(b):
        pltpu.make_async_copy(x_hbm.at[pidx[b].at[0, 0]], rows[b], gsem[b]).wait()

    def do_scatter(b):
        pltpu.sync_copy(rows[b], acc.at[didx[b]], add=True)

    def copy_didx(b):
        for g in range(CH // 16):
            didx[b][pl.ds(16 * g, 16)] = pidx[b][0, 1, pl.ds(16 * g, 16)]

    def scale(b):
        def grp(g, carry):
            wvec = wring[b][0, pl.ds(g * 16, 16)]
            r0 = g * 16
            for lane in range(16):
                wspl = jnp.full((16,), wvec[lane], jnp.float32)
                for j in range(8):
                    rows[b][r0 + lane, pl.ds(16 * j, 16)] = (
                        rows[b][r0 + lane, pl.ds(16 * j, 16)] * wspl)
            return carry

        lax.fori_loop(0, CH // 16, grp, 0)

    def zero_row(r, carry):
        for j in range(8):
            r2_v[r, pl.ds(16 * j, 16)] = jnp.zeros((16,), jnp.float32)
        return carry

    lax.fori_loop(0, CH, zero_row, 0)
    abase = sid * RPT
    for k in range(RPT // CH):
        pltpu.sync_copy(r2_v, acc.at[pl.ds(abase + k * CH, CH)])

    plsc.subcore_barrier()

    def phase(c, b, bn, do_gather=True, do_fetch=True):
        sync_fetch(c, b)
        copy_didx(b)
        start_gather(b)
        wait_gather(b)
        scale(b)
        do_scatter(b)

    phase(0, 0, 2)
    phase(1, 1, 3)

    def main(s, carry):
        c0 = 2 + 4 * s
        phase(c0 + 0, 2, 0)
        phase(c0 + 1, 3, 1)
        phase(c0 + 2, 0, 2)
        phase(c0 + 3, 1, 3)
        return carry

    lax.fori_loop(0, (NCHUNK - 4) // 4, main, 0)

    phase(NCHUNK - 2, 2, 0, do_gather=False, do_fetch=False)
    phase(NCHUNK - 1, 3, 1, do_gather=False, do_fetch=False)
    plsc.subcore_barrier()

    pltpu.sync_copy(acc.at[pl.ds(abase, RPT)],
                    part_hbm.at[pl.ds(cid * NP + abase, RPT)])


@jax.jit
def _sc_layer(x, packed, w):
    mesh = plsc.VectorSubcoreMesh(core_axis_name="c", subcore_axis_name="s")
    return pl.kernel(
        _sc_layer_body,
        out_type=jax.ShapeDtypeStruct((NC * NP, D), jnp.float32),
        mesh=mesh,
        scratch_types=(
            [pltpu.VMEM((CH, D), jnp.float32)] * 4
            + [pltpu.VMEM((1, 2, CH), jnp.int32)] * 4
            + [pltpu.VMEM((1, CH), jnp.float32)] * 4
            + [pltpu.VMEM((CH,), jnp.int32)] * 4
            + [pltpu.VMEM_SHARED((NP, D), jnp.float32)]
            + [pltpu.SemaphoreType.DMA] * 8
        ),
    )(x, packed, w)


def _combine_body(p0_ref, p1_ref, a_ref, x_ref, ao_ref):
    s = p0_ref[...] + p1_ref[...]
    x_ref[...] = s
    ao_ref[...] = a_ref[...] + s


def _final_body(p0_ref, p1_ref, a_ref, m_ref):
    m_ref[...] = (a_ref[...] + p0_ref[...] + p1_ref[...]) * 0.25


_BLK = 1280


def _row_spec():
    return pl.BlockSpec((_BLK, D), lambda i: (i, 0))


@jax.jit
def _combine(p0, p1, a):
    return pl.pallas_call(
        _combine_body,
        grid=(NP // _BLK,),
        in_specs=[_row_spec(), _row_spec(), _row_spec()],
        out_specs=[_row_spec(), _row_spec()],
        out_shape=[jax.ShapeDtypeStruct((NP, D), jnp.float32)] * 2,
    )(p0, p1, a)


@jax.jit
def _finalize(p0, p1, a):
    return pl.pallas_call(
        _final_body,
        grid=(NP // _BLK,),
        in_specs=[_row_spec(), _row_spec(), _row_spec()],
        out_specs=_row_spec(),
        out_shape=jax.ShapeDtypeStruct((NP, D), jnp.float32),
    )(p0, p1, a)


def kernel(user_emb, item_emb, edge_weight, edge_index):
    x0 = jnp.pad(jnp.concatenate([user_emb, item_emb], axis=0),
                 ((0, NP - NN), (0, 0)))
    pad = EPAD - NE
    src = jnp.pad(edge_index[1], (0, pad)).reshape(-1, CH)
    dst = jnp.pad(edge_index[0], (0, pad)).reshape(-1, CH)
    w = jnp.pad(edge_weight, (0, pad)).reshape(-1, CH)
    packed = jnp.stack([src, dst], axis=1)

    x = x0
    acc = x0
    for layer in range(NLAYER):
        part = _sc_layer(x, packed, w)
        p0 = part[:NP]
        p1 = part[NP:]
        if layer < NLAYER - 1:
            x, acc = _combine(p0, p1, acc)
        else:
            mean = _finalize(p0, p1, acc)
    return (mean[:NU], mean[NU:NN])


# speedup vs baseline: 1.3371x; 1.3371x over previous
import jax
import jax.numpy as jnp
from jax import lax
from jax.experimental import pallas as pl
from jax.experimental.pallas import tpu as pltpu
from jax.experimental.pallas import tpu_sc as plsc

NU = 4000
NI = 6000
NN = NU + NI
NE = 320000
D = 128
NLAYER = 3

NC = 2
NS = 16
NW = NC * NS
CH = 80
EPT = 10240
EPAD = NW * EPT
NCHUNK = EPT // CH
NP = 10240
RPT = NP // NS


def _sc_layer_body(x_hbm, packed_hbm, w_hbm, part_hbm,
                   r0_v, r1_v, r2_v, r3_v,
                   pa_v, pb_v, wa_v, wb_v,
                   d0_v, d1_v, d2_v, d3_v, acc,
                   g0, g1, g2, g3, s0, s1, s2, s3, fsm):
    cid = lax.axis_index("c")
    sid = lax.axis_index("s")
    rows = (r0_v, r1_v, r2_v, r3_v)
    didx = (d0_v, d1_v, d2_v, d3_v)
    gsem = (g0, g1, g2, g3)
    ssem = (s0, s1, s2, s3)
    wid = cid * NS + sid
    cbase = wid * NCHUNK

    def scale(i, pbuf, wbuf):
        def grp(g, carry):
            wvec = wbuf[i, pl.ds(g * 16, 16)]
            r0 = g * 16
            for lane in range(16):
                wspl = jnp.full((16,), wvec[lane], jnp.float32)
                for j in range(8):
                    rows[i][r0 + lane, pl.ds(16 * j, 16)] = (
                        rows[i][r0 + lane, pl.ds(16 * j, 16)] * wspl)
            return carry

        lax.fori_loop(0, CH // 16, grp, 0)

    def zero_row(r, carry):
        for j in range(8):
            r2_v[r, pl.ds(16 * j, 16)] = jnp.zeros((16,), jnp.float32)
        return carry

    lax.fori_loop(0, CH, zero_row, 0)
    abase = sid * RPT
    for k in range(RPT // CH):
        pltpu.sync_copy(r2_v, acc.at[pl.ds(abase + k * CH, CH)])

    pltpu.sync_copy(packed_hbm.at[pl.ds(cbase, 4)], pa_v)
    pltpu.sync_copy(w_hbm.at[pl.ds(cbase, 4)], wa_v)
    plsc.subcore_barrier()

    def subiter(c0, pbuf, wbuf, pnext, wnext):
        cf = jnp.minimum(c0 + 4, NCHUNK - 4)
        fp = pltpu.async_copy(packed_hbm.at[pl.ds(cbase + cf, 4)], pnext, fsm)
        fw = pltpu.async_copy(w_hbm.at[pl.ds(cbase + cf, 4)], wnext, fsm)
        gd = [pltpu.async_copy(x_hbm.at[pbuf.at[i, 0]], rows[i], gsem[i])
              for i in range(4)]
        sd = []
        for i in range(4):
            gd[i].wait()
            for g in range(CH // 16):
                didx[i][pl.ds(16 * g, 16)] = pbuf[i, 1, pl.ds(16 * g, 16)]
            scale(i, pbuf, wbuf)
            sd.append(pltpu.async_copy(rows[i], acc.at[didx[i]], ssem[i],
                                       add=True))
        for d in sd:
            d.wait()
        fp.wait()
        fw.wait()

    def body(s2, carry):
        c0 = 8 * s2
        subiter(c0, pa_v, wa_v, pb_v, wb_v)
        subiter(c0 + 4, pb_v, wb_v, pa_v, wa_v)
        return carry

    lax.fori_loop(0, NCHUNK // 8, body, 0)
    plsc.subcore_barrier()

    pltpu.sync_copy(acc.at[pl.ds(abase, RPT)],
                    part_hbm.at[pl.ds(cid * NP + abase, RPT)])


@jax.jit
def _sc_layer(x, packed, w):
    mesh = plsc.VectorSubcoreMesh(core_axis_name="c", subcore_axis_name="s")
    return pl.kernel(
        _sc_layer_body,
        out_type=jax.ShapeDtypeStruct((NC * NP, D), jnp.float32),
        mesh=mesh,
        scratch_types=(
            [pltpu.VMEM((CH, D), jnp.float32)] * 4
            + [pltpu.VMEM((4, 2, CH), jnp.int32)] * 2
            + [pltpu.VMEM((4, CH), jnp.float32)] * 2
            + [pltpu.VMEM((CH,), jnp.int32)] * 4
            + [pltpu.VMEM_SHARED((NP, D), jnp.float32)]
            + [pltpu.SemaphoreType.DMA] * 9
        ),
    )(x, packed, w)


def _combine_body(p0_ref, p1_ref, a_ref, x_ref, ao_ref):
    s = p0_ref[...] + p1_ref[...]
    x_ref[...] = s
    ao_ref[...] = a_ref[...] + s


def _final_body(p0_ref, p1_ref, a_ref, m_ref):
    m_ref[...] = (a_ref[...] + p0_ref[...] + p1_ref[...]) * 0.25


_BLK = 1280


def _row_spec():
    return pl.BlockSpec((_BLK, D), lambda i: (i, 0))


@jax.jit
def _combine(p0, p1, a):
    return pl.pallas_call(
        _combine_body,
        grid=(NP // _BLK,),
        in_specs=[_row_spec(), _row_spec(), _row_spec()],
        out_specs=[_row_spec(), _row_spec()],
        out_shape=[jax.ShapeDtypeStruct((NP, D), jnp.float32)] * 2,
    )(p0, p1, a)


@jax.jit
def _finalize(p0, p1, a):
    return pl.pallas_call(
        _final_body,
        grid=(NP // _BLK,),
        in_specs=[_row_spec(), _row_spec(), _row_spec()],
        out_specs=_row_spec(),
        out_shape=jax.ShapeDtypeStruct((NP, D), jnp.float32),
    )(p0, p1, a)


def kernel(user_emb, item_emb, edge_weight, edge_index):
    x0 = jnp.pad(jnp.concatenate([user_emb, item_emb], axis=0),
                 ((0, NP - NN), (0, 0)))
    pad = EPAD - NE
    src = jnp.pad(edge_index[1], (0, pad)).reshape(-1, CH)
    dst = jnp.pad(edge_index[0], (0, pad)).reshape(-1, CH)
    w = jnp.pad(edge_weight, (0, pad)).reshape(-1, CH)
    packed = jnp.stack([src, dst], axis=1)

    x = x0
    acc = x0
    for layer in range(NLAYER):
        part = _sc_layer(x, packed, w)
        p0 = part[:NP]
        p1 = part[NP:]
        if layer < NLAYER - 1:
            x, acc = _combine(p0, p1, acc)
        else:
            mean = _finalize(p0, p1, acc)
    return (mean[:NU], mean[NU:NN])


# speedup vs baseline: 1.5610x; 1.1675x over previous
import jax
import jax.numpy as jnp
from jax import lax
from jax.experimental import pallas as pl
from jax.experimental.pallas import tpu as pltpu
from jax.experimental.pallas import tpu_sc as plsc

NU = 4000
NI = 6000
NN = NU + NI
NE = 320000
D = 128
NLAYER = 3

NC = 2
NS = 16
NW = NC * NS
CH = 80
EPT = 10240
EPAD = NW * EPT
NCHUNK = EPT // CH
NCK0 = 192
NCK1 = 64
NP = 10240
RPT = NP // NS


def _sc_layer_body(x_hbm, packed_hbm, w_hbm, part_hbm,
                   r0_v, r1_v, r2_v, r3_v,
                   pa_v, pb_v, wa_v, wb_v,
                   d0_v, d1_v, d2_v, d3_v, acc,
                   g0, g1, g2, g3, s0, s1, s2, s3, fsm):
    cid = lax.axis_index("c")
    sid = lax.axis_index("s")
    rows = (r0_v, r1_v, r2_v, r3_v)
    didx = (d0_v, d1_v, d2_v, d3_v)
    gsem = (g0, g1, g2, g3)
    ssem = (s0, s1, s2, s3)
    nck = jnp.where(cid == 0, NCK0, NCK1)
    cbase = jnp.where(cid == 0, sid * NCK0, NS * NCK0 + sid * NCK1)

    def scale(i, pbuf, wbuf):
        def grp(g, carry):
            wvec = wbuf[i, pl.ds(g * 16, 16)]
            r0 = g * 16
            for lane in range(16):
                wspl = jnp.full((16,), wvec[lane], jnp.float32)
                for j in range(8):
                    rows[i][r0 + lane, pl.ds(16 * j, 16)] = (
                        rows[i][r0 + lane, pl.ds(16 * j, 16)] * wspl)
            return carry

        lax.fori_loop(0, CH // 16, grp, 0)

    def zero_row(r, carry):
        for j in range(8):
            r2_v[r, pl.ds(16 * j, 16)] = jnp.zeros((16,), jnp.float32)
        return carry

    lax.fori_loop(0, CH, zero_row, 0)
    abase = sid * RPT
    for k in range(RPT // CH):
        pltpu.sync_copy(r2_v, acc.at[pl.ds(abase + k * CH, CH)])

    pltpu.sync_copy(packed_hbm.at[pl.ds(cbase, 4)], pa_v)
    pltpu.sync_copy(w_hbm.at[pl.ds(cbase, 4)], wa_v)
    plsc.subcore_barrier()

    def subiter(c0, pbuf, wbuf, pnext, wnext):
        cf = jnp.minimum(c0 + 4, nck - 4)
        fp = pltpu.async_copy(packed_hbm.at[pl.ds(cbase + cf, 4)], pnext, fsm)
        fw = pltpu.async_copy(w_hbm.at[pl.ds(cbase + cf, 4)], wnext, fsm)
        gd = [pltpu.async_copy(x_hbm.at[pbuf.at[i, 0]], rows[i], gsem[i])
              for i in range(4)]
        sd = []
        for i in range(4):
            gd[i].wait()
            for g in range(CH // 16):
                didx[i][pl.ds(16 * g, 16)] = pbuf[i, 1, pl.ds(16 * g, 16)]
            scale(i, pbuf, wbuf)
            sd.append(pltpu.async_copy(rows[i], acc.at[didx[i]], ssem[i],
                                       add=True))
        for d in sd:
            d.wait()
        fp.wait()
        fw.wait()

    def body(s2, carry):
        c0 = 8 * s2
        subiter(c0, pa_v, wa_v, pb_v, wb_v)
        subiter(c0 + 4, pb_v, wb_v, pa_v, wa_v)
        return carry

    lax.fori_loop(0, nck // 8, body, 0)
    plsc.subcore_barrier()

    pltpu.sync_copy(acc.at[pl.ds(abase, RPT)],
                    part_hbm.at[pl.ds(cid * NP + abase, RPT)])


@jax.jit
def _sc_layer(x, packed, w):
    mesh = plsc.VectorSubcoreMesh(core_axis_name="c", subcore_axis_name="s")
    return pl.kernel(
        _sc_layer_body,
        out_type=jax.ShapeDtypeStruct((NC * NP, D), jnp.float32),
        mesh=mesh,
        scratch_types=(
            [pltpu.VMEM((CH, D), jnp.float32)] * 4
            + [pltpu.VMEM((4, 2, CH), jnp.int32)] * 2
            + [pltpu.VMEM((4, CH), jnp.float32)] * 2
            + [pltpu.VMEM((CH,), jnp.int32)] * 4
            + [pltpu.VMEM_SHARED((NP, D), jnp.float32)]
            + [pltpu.SemaphoreType.DMA] * 9
        ),
    )(x, packed, w)


def _combine_body(p0_ref, p1_ref, a_ref, x_ref, ao_ref):
    s = p0_ref[...] + p1_ref[...]
    x_ref[...] = s
    ao_ref[...] = a_ref[...] + s


def _final_body(p0_ref, p1_ref, a_ref, m_ref):
    m_ref[...] = (a_ref[...] + p0_ref[...] + p1_ref[...]) * 0.25


_BLK = 1280


def _row_spec():
    return pl.BlockSpec((_BLK, D), lambda i: (i, 0))


@jax.jit
def _combine(p0, p1, a):
    return pl.pallas_call(
        _combine_body,
        grid=(NP // _BLK,),
        in_specs=[_row_spec(), _row_spec(), _row_spec()],
        out_specs=[_row_spec(), _row_spec()],
        out_shape=[jax.ShapeDtypeStruct((NP, D), jnp.float32)] * 2,
    )(p0, p1, a)


@jax.jit
def _finalize(p0, p1, a):
    return pl.pallas_call(
        _final_body,
        grid=(NP // _BLK,),
        in_specs=[_row_spec(), _row_spec(), _row_spec()],
        out_specs=_row_spec(),
        out_shape=jax.ShapeDtypeStruct((NP, D), jnp.float32),
    )(p0, p1, a)


def kernel(user_emb, item_emb, edge_weight, edge_index):
    x0 = jnp.pad(jnp.concatenate([user_emb, item_emb], axis=0),
                 ((0, NP - NN), (0, 0)))
    pad = EPAD - NE
    src = jnp.pad(edge_index[1], (0, pad)).reshape(-1, CH)
    dst = jnp.pad(edge_index[0], (0, pad)).reshape(-1, CH)
    w = jnp.pad(edge_weight, (0, pad)).reshape(-1, CH)
    packed = jnp.stack([src, dst], axis=1)

    x = x0
    acc = x0
    for layer in range(NLAYER):
        part = _sc_layer(x, packed, w)
        p0 = part[:NP]
        p1 = part[NP:]
        if layer < NLAYER - 1:
            x, acc = _combine(p0, p1, acc)
        else:
            mean = _finalize(p0, p1, acc)
    return (mean[:NU], mean[NU:NN])
